# baseline (speedup 1.0000x reference)
import functools

import jax
import jax.numpy as jnp
from jax import lax
from jax.experimental import pallas as pl
from jax.experimental.pallas import tpu as pltpu
from jax.experimental.pallas import tpu_sc as plsc

B, N, M, O, E, F, K = 4, 65536, 4096, 16, 64, 128, 7
KF = K * F
NC, NS = 2, 16
NW = NC * NS
QT = (B * M) // NW
CQ = 8
RC = CQ * O
NCHUNK = QT // CQ


def _sc_body(x_hbm, xe_hbm, ye_hbm, i_hbm, lt_hbm, out_hbm,
             ye_v, lt_v, i_v, w_v,
             idx_v0, xe_v0, x_v0, out_v0, sem_xe0, sem_x0, sem_o0,
             idx_v1, xe_v1, x_v1, out_v1, sem_xe1, sem_x1, sem_o1):
    cid = lax.axis_index("c")
    sid = lax.axis_index("s")
    wid = sid * NC + cid
    qw = pl.multiple_of(wid * QT, QT)
    bN = (qw // M) * N

    pltpu.sync_copy(ye_hbm.at[pl.ds(qw, QT)], ye_v)
    pltpu.sync_copy(lt_hbm.at[pl.ds(qw, QT)], lt_v)
    pltpu.sync_copy(i_hbm.at[pl.ds(qw, QT)], i_v)

    lanes = lax.broadcasted_iota(jnp.int32, (O,), 0)

    def issue_gathers(c, idx_v, xe_v, x_v, sem_xe, sem_x):
        def idx_body(q, _):
            idx_v[pl.ds(q * O, O)] = i_v[c * CQ + q, :] + bN
            return 0
        lax.fori_loop(0, CQ, idx_body, 0)
        pltpu.async_copy(xe_hbm.at[idx_v], xe_v, sem_xe)
        pltpu.async_copy(x_hbm.at[idx_v], x_v, sem_x)

    def compute_chunk(c, xe_v, x_v, out_v):
        def q_body(q, _):
            qrow = c * CQ + q
            qsplat = jnp.full((O,), qrow, jnp.int32)

            ye_r = [ye_v[qrow, pl.ds(u * 16, 16)] for u in range(4)]

            def o_dist(o, dvec):
                acc = jnp.zeros((O,), jnp.float32)
                for u in range(4):
                    diff = xe_v[q * O + o, pl.ds(u * 16, 16)] - ye_r[u]
                    acc = acc + diff * diff
                return jnp.where(lanes == o, jnp.sum(acc), dvec)
            d = lax.fori_loop(0, O, o_dist, jnp.zeros((O,), jnp.float32))

            temp = jnp.exp(plsc.load_gather(lt_v, [qsplat]))
            logits = -d / temp
            s = jnp.exp(logits - jnp.max(logits))
            w = s / jnp.sum(s)
            for k in range(K):
                w_v[pl.ds(k * O, O)] = w
                if k + 1 < K:
                    w2 = w * jnp.maximum(1.0 - w, 1e-7)
                    w = w2 / jnp.sum(w2)

            def o_body(o, accs):
                xr = [plsc.bitcast(x_v[q * O + o, pl.ds(j * 16, 16)],
                                   jnp.bfloat16) for j in range(4)]
                out = []
                for t in range(K):
                    ws = plsc.load_gather(
                        w_v, [jnp.full((O,), t * O + o, jnp.int32)])
                    wsb = plsc.pack(ws, ws, format=plsc.PackFormat.INTERLEAVED)
                    for j in range(4):
                        out.append(accs[t * 4 + j] + wsb * xr[j])
                return tuple(out)
            accs = lax.fori_loop(
                0, O, o_body,
                tuple(jnp.zeros((32,), jnp.bfloat16) for _ in range(K * 4)))
            lanes2 = lanes * 2
            for t in range(K):
                for j in range(4):
                    lo, hi = plsc.unpack(accs[t * 4 + j],
                                         format=plsc.PackFormat.INTERLEAVED)
                    base = jnp.full((O,), q * KF + t * F + j * 32,
                                    jnp.int32) + lanes2
                    plsc.store_scatter(out_v, [base], lo)
                    plsc.store_scatter(out_v, [base + 1], hi)
            return 0
        lax.fori_loop(0, CQ, q_body, 0)

    def wait_gathers(idx_v, xe_v, x_v, sem_xe, sem_x):
        pltpu.make_async_copy(xe_hbm.at[idx_v], xe_v, sem_xe).wait()
        pltpu.make_async_copy(x_hbm.at[idx_v], x_v, sem_x).wait()

    def store_out(c, out_v, sem_o):
        pltpu.async_copy(
            out_v, out_hbm.at[pl.ds((qw + c * CQ) * KF, CQ * KF)], sem_o)

    def wait_store(c, out_v, sem_o):
        pltpu.make_async_copy(
            out_v, out_hbm.at[pl.ds((qw + c * CQ) * KF, CQ * KF)], sem_o).wait()

    issue_gathers(0, idx_v0, xe_v0, x_v0, sem_xe0, sem_x0)

    def pair_body(p, _):
        c0 = p * 2
        c1 = c0 + 1
        issue_gathers(c1, idx_v1, xe_v1, x_v1, sem_xe1, sem_x1)
        wait_gathers(idx_v0, xe_v0, x_v0, sem_xe0, sem_x0)

        @pl.when(p > 0)
        def _():
            wait_store(c0 - 2, out_v0, sem_o0)
        compute_chunk(c0, xe_v0, x_v0, out_v0)
        store_out(c0, out_v0, sem_o0)

        @pl.when(p < NCHUNK // 2 - 1)
        def _():
            issue_gathers(c0 + 2, idx_v0, xe_v0, x_v0, sem_xe0, sem_x0)
        wait_gathers(idx_v1, xe_v1, x_v1, sem_xe1, sem_x1)

        @pl.when(p > 0)
        def _():
            wait_store(c1 - 2, out_v1, sem_o1)
        compute_chunk(c1, xe_v1, x_v1, out_v1)
        store_out(c1, out_v1, sem_o1)
        return 0
    lax.fori_loop(0, NCHUNK // 2, pair_body, 0)
    wait_store(NCHUNK - 2, out_v0, sem_o0)
    wait_store(NCHUNK - 1, out_v1, sem_o1)


@functools.partial(jax.jit, static_argnums=())
def _run(x2, xe2, ye2, i2, lt2):
    f = pl.kernel(
        _sc_body,
        out_type=jax.ShapeDtypeStruct((B * M * KF,), jnp.float32),
        mesh=plsc.VectorSubcoreMesh(
            core_axis_name="c", subcore_axis_name="s",
            num_cores=NC, num_subcores=NS),
        compiler_params=pltpu.CompilerParams(
            needs_layout_passes=False, use_tc_tiling_on_sc=False),
        scratch_types=[
            pltpu.VMEM((QT, E), jnp.float32),
            pltpu.VMEM((QT,), jnp.float32),
            pltpu.VMEM((QT, O), jnp.int32),
            pltpu.VMEM((CQ * K * O,), jnp.float32),
            pltpu.VMEM((RC,), jnp.int32),
            pltpu.VMEM((RC, E), jnp.float32),
            pltpu.VMEM((RC, F // 2), jnp.float32),
            pltpu.VMEM((CQ * KF,), jnp.float32),
            pltpu.SemaphoreType.DMA,
            pltpu.SemaphoreType.DMA,
            pltpu.SemaphoreType.DMA,
            pltpu.VMEM((RC,), jnp.int32),
            pltpu.VMEM((RC, E), jnp.float32),
            pltpu.VMEM((RC, F // 2), jnp.float32),
            pltpu.VMEM((CQ * KF,), jnp.float32),
            pltpu.SemaphoreType.DMA,
            pltpu.SemaphoreType.DMA,
            pltpu.SemaphoreType.DMA,
        ],
    )
    return f(x2, xe2, ye2, i2, lt2)


def kernel(x, xe, ye, I, log_temp):
    xb = x.reshape(B * N, F).astype(jnp.bfloat16)
    x2 = jax.lax.bitcast_convert_type(
        xb.reshape(B * N, F // 2, 2), jnp.float32)
    xe2 = xe.reshape(B * N, E)
    ye2 = ye.reshape(B * M, E)
    i2 = I.astype(jnp.int32).reshape(B * M, O)
    lt2 = log_temp.astype(jnp.float32).reshape(B * M)
    out = _run(x2, xe2, ye2, i2, lt2)
    return out.reshape(B, M, KF)


# speedup vs baseline: 2.3577x; 2.3577x over previous
import functools

import jax
import jax.numpy as jnp
from jax import lax
from jax.experimental import pallas as pl
from jax.experimental.pallas import tpu as pltpu
from jax.experimental.pallas import tpu_sc as plsc

B, N, M, O, E, F, K = 4, 65536, 4096, 16, 64, 128, 7
KF = K * F
NC, NS = 2, 16
NW = NC * NS
QT = (B * M) // NW
CQ = 8
RC = CQ * O
NCHUNK = QT // CQ


def _sc_body(x_hbm, xe_hbm, ye_hbm, i_hbm, lt_hbm, out_hbm,
             ye_v, lt_v, i_v, w_v,
             idx_v0, xe_v0, x_v0, out_v0, sem_xe0, sem_x0, sem_o0,
             idx_v1, xe_v1, x_v1, out_v1, sem_xe1, sem_x1, sem_o1):
    cid = lax.axis_index("c")
    sid = lax.axis_index("s")
    wid = sid * NC + cid
    qw = pl.multiple_of(wid * QT, QT)
    bN = (qw // M) * N

    pltpu.sync_copy(ye_hbm.at[pl.ds(qw, QT)], ye_v)
    pltpu.sync_copy(lt_hbm.at[pl.ds(qw, QT)], lt_v)
    pltpu.sync_copy(i_hbm.at[pl.ds(qw, QT)], i_v)

    lanes = lax.broadcasted_iota(jnp.int32, (O,), 0)

    def issue_gathers(c, idx_v, xe_v, x_v, sem_xe, sem_x):
        def idx_body(q, _):
            idx_v[pl.ds(q * O, O)] = i_v[c * CQ + q, :] + bN
            return 0
        lax.fori_loop(0, CQ, idx_body, 0)
        pltpu.async_copy(xe_hbm.at[idx_v], xe_v, sem_xe)
        pltpu.async_copy(x_hbm.at[idx_v], x_v, sem_x)

    def compute_chunk(c, xe_v, x_v, out_v):
        def q_body(q, _):
            qrow = c * CQ + q
            qsplat = jnp.full((O,), qrow, jnp.int32)

            ye_r = [ye_v[qrow, pl.ds(u * 16, 16)] for u in range(4)]

            def o_dist(o, dvec):
                acc = jnp.zeros((O,), jnp.float32)
                for u in range(4):
                    diff = xe_v[q * O + o, pl.ds(u * 16, 16)] - ye_r[u]
                    acc = acc + diff * diff
                return jnp.where(lanes == o, jnp.sum(acc), dvec)
            d = lax.fori_loop(0, O, o_dist, jnp.zeros((O,), jnp.float32))

            temp = jnp.exp(plsc.load_gather(lt_v, [qsplat]))
            logits = -d / temp
            s = jnp.exp(logits - jnp.max(logits))
            w = s / jnp.sum(s)
            for k in range(K):
                w_v[pl.ds(k * O, O)] = w
                if k + 1 < K:
                    w2 = w * jnp.maximum(1.0 - w, 1e-7)
                    w = w2 / jnp.sum(w2)

            for k0, kn in ((0, 4), (4, 3)):
                def o_body(o, accs):
                    xr = [x_v[q * O + o, pl.ds(j * 16, 16)] for j in range(8)]
                    out = []
                    for t in range(kn):
                        ws = plsc.load_gather(
                            w_v, [jnp.full((O,), (k0 + t) * O + o, jnp.int32)])
                        for j in range(8):
                            out.append(accs[t * 8 + j] + ws * xr[j])
                    return tuple(out)
                accs = lax.fori_loop(
                    0, O, o_body,
                    tuple(jnp.zeros((O,), jnp.float32) for _ in range(kn * 8)))
                for t in range(kn):
                    for j in range(8):
                        out_v[q, pl.ds((k0 + t) * F + j * 16, 16)] = accs[t * 8 + j]
            return 0
        lax.fori_loop(0, CQ, q_body, 0)

    def wait_gathers(idx_v, xe_v, x_v, sem_xe, sem_x):
        pltpu.make_async_copy(xe_hbm.at[idx_v], xe_v, sem_xe).wait()
        pltpu.make_async_copy(x_hbm.at[idx_v], x_v, sem_x).wait()

    def store_out(c, out_v, sem_o):
        pltpu.async_copy(out_v, out_hbm.at[pl.ds(qw + c * CQ, CQ)], sem_o)

    def wait_store(c, out_v, sem_o):
        pltpu.make_async_copy(
            out_v, out_hbm.at[pl.ds(qw + c * CQ, CQ)], sem_o).wait()

    issue_gathers(0, idx_v0, xe_v0, x_v0, sem_xe0, sem_x0)

    def pair_body(p, _):
        c0 = p * 2
        c1 = c0 + 1
        issue_gathers(c1, idx_v1, xe_v1, x_v1, sem_xe1, sem_x1)
        wait_gathers(idx_v0, xe_v0, x_v0, sem_xe0, sem_x0)

        @pl.when(p > 0)
        def _():
            wait_store(c0 - 2, out_v0, sem_o0)
        compute_chunk(c0, xe_v0, x_v0, out_v0)
        store_out(c0, out_v0, sem_o0)

        @pl.when(p < NCHUNK // 2 - 1)
        def _():
            issue_gathers(c0 + 2, idx_v0, xe_v0, x_v0, sem_xe0, sem_x0)
        wait_gathers(idx_v1, xe_v1, x_v1, sem_xe1, sem_x1)

        @pl.when(p > 0)
        def _():
            wait_store(c1 - 2, out_v1, sem_o1)
        compute_chunk(c1, xe_v1, x_v1, out_v1)
        store_out(c1, out_v1, sem_o1)
        return 0
    lax.fori_loop(0, NCHUNK // 2, pair_body, 0)
    wait_store(NCHUNK - 2, out_v0, sem_o0)
    wait_store(NCHUNK - 1, out_v1, sem_o1)


@functools.partial(jax.jit, static_argnums=())
def _run(x2, xe2, ye2, i2, lt2):
    f = pl.kernel(
        _sc_body,
        out_type=jax.ShapeDtypeStruct((B * M, KF), jnp.float32),
        mesh=plsc.VectorSubcoreMesh(
            core_axis_name="c", subcore_axis_name="s",
            num_cores=NC, num_subcores=NS),
        compiler_params=pltpu.CompilerParams(
            needs_layout_passes=False, use_tc_tiling_on_sc=False),
        scratch_types=[
            pltpu.VMEM((QT, E), jnp.float32),
            pltpu.VMEM((QT,), jnp.float32),
            pltpu.VMEM((QT, O), jnp.int32),
            pltpu.VMEM((K * O,), jnp.float32),
            pltpu.VMEM((RC,), jnp.int32),
            pltpu.VMEM((RC, E), jnp.float32),
            pltpu.VMEM((RC, F), jnp.float32),
            pltpu.VMEM((CQ, KF), jnp.float32),
            pltpu.SemaphoreType.DMA,
            pltpu.SemaphoreType.DMA,
            pltpu.SemaphoreType.DMA,
            pltpu.VMEM((RC,), jnp.int32),
            pltpu.VMEM((RC, E), jnp.float32),
            pltpu.VMEM((RC, F), jnp.float32),
            pltpu.VMEM((CQ, KF), jnp.float32),
            pltpu.SemaphoreType.DMA,
            pltpu.SemaphoreType.DMA,
            pltpu.SemaphoreType.DMA,
        ],
    )
    return f(x2, xe2, ye2, i2, lt2)


def kernel(x, xe, ye, I, log_temp):
    x2 = x.reshape(B * N, F)
    xe2 = xe.reshape(B * N, E)
    ye2 = ye.reshape(B * M, E)
    i2 = I.astype(jnp.int32).reshape(B * M, O)
    lt2 = log_temp.astype(jnp.float32).reshape(B * M)
    out = _run(x2, xe2, ye2, i2, lt2)
    return out.reshape(B, M, KF)
